# Initial kernel scaffold; baseline (speedup 1.0000x reference)
import jax
import jax.numpy as jnp
from jax.experimental import pallas as pl


def kernel(batch, key_length, query_length, table):
    raise NotImplementedError("write your pallas kernel here")


# speedup vs baseline: 64.9048x; 64.9048x over previous
import math

import jax
import jax.numpy as jnp
from jax.experimental import pallas as pl
from jax.experimental.pallas import tpu as pltpu

_B = 2
_H = 16
_K = 2048
_Q = 2048
_NB = 32
_MD = 128
_LINE = 4096
_TK = 256


def _line_kernel(table_ref, gline_ref):
    j = jax.lax.broadcasted_iota(jnp.int32, (1, _LINE), 1)
    d = 2047 - j
    nb = _NB // 2
    rb = jnp.where(d > 0, nb, 0)
    ad = jnp.abs(d)
    max_exact = nb // 2
    is_small = ad < max_exact
    rp_safe = jnp.maximum(ad, 1).astype(jnp.float32)
    if_large = max_exact + (
        jnp.log(rp_safe / max_exact) / math.log(_MD / max_exact) * (nb - max_exact)
    ).astype(jnp.int32)
    if_large = jnp.minimum(if_large, nb - 1)
    bucket = rb + jnp.where(is_small, ad, if_large)
    rows = jax.lax.broadcasted_iota(jnp.int32, (_NB, _LINE), 0)
    onehot = (rows == bucket).astype(jnp.float32)
    gline_ref[:, 0, :] = jax.lax.dot_general(
        table_ref[...], onehot, (((0,), (0,)), ((), ())),
        preferred_element_type=jnp.float32)


def _bcast_kernel(gline_ref, out_ref):
    kt = pl.program_id(1)
    big = jnp.broadcast_to(gline_ref[0], (_TK, _LINE))
    rolled = pltpu.roll(big, _LINE // 2 + 1, 1, stride=1, stride_axis=0)
    rolled = pltpu.roll(rolled, kt * _TK, 1)
    out_ref[...] = jnp.broadcast_to(rolled[None, None, :, :_Q], (_B, 1, _TK, _Q))


def kernel(batch, key_length, query_length, table):
    gline = pl.pallas_call(
        _line_kernel,
        out_shape=jax.ShapeDtypeStruct((_H, 1, _LINE), jnp.float32),
    )(table)
    out = pl.pallas_call(
        _bcast_kernel,
        grid=(_H, _K // _TK),
        in_specs=[pl.BlockSpec((1, 1, _LINE), lambda h, kt: (h, 0, 0))],
        out_specs=pl.BlockSpec((_B, 1, _TK, _Q), lambda h, kt: (0, h, kt, 0)),
        out_shape=jax.ShapeDtypeStruct((_B, _H, _K, _Q), jnp.float32),
    )(gline)
    return out


# speedup vs baseline: 96.2560x; 1.4830x over previous
import math

import jax
import jax.numpy as jnp
from jax.experimental import pallas as pl
from jax.experimental.pallas import tpu as pltpu

_B = 2
_H = 16
_K = 2048
_Q = 2048
_NB = 32
_MD = 128
_LINE = 4096
_TK = 512


def _line_kernel(table_ref, gline_ref):
    j = jax.lax.broadcasted_iota(jnp.int32, (1, _LINE), 1)
    d = 2047 - j
    nb = _NB // 2
    rb = jnp.where(d > 0, nb, 0)
    ad = jnp.abs(d)
    max_exact = nb // 2
    is_small = ad < max_exact
    rp_safe = jnp.maximum(ad, 1).astype(jnp.float32)
    if_large = max_exact + (
        jnp.log(rp_safe / max_exact) / math.log(_MD / max_exact) * (nb - max_exact)
    ).astype(jnp.int32)
    if_large = jnp.minimum(if_large, nb - 1)
    bucket = rb + jnp.where(is_small, ad, if_large)
    rows = jax.lax.broadcasted_iota(jnp.int32, (_NB, _LINE), 0)
    onehot = (rows == bucket).astype(jnp.float32)
    gline_ref[:, 0, :] = jax.lax.dot_general(
        table_ref[...], onehot, (((0,), (0,)), ((), ())),
        preferred_element_type=jnp.float32)


def _bcast_kernel(gline_ref, out_ref):
    kt = pl.program_id(1)
    big = jnp.broadcast_to(gline_ref[0], (_TK, _LINE))
    rolled = pltpu.roll(big, _LINE // 2 + 1, 1, stride=1, stride_axis=0)
    rolled = pltpu.roll(rolled, kt * _TK, 1)
    out_ref[...] = jnp.broadcast_to(rolled[None, None, :, :_Q], (_B, 1, _TK, _Q))


def kernel(batch, key_length, query_length, table):
    gline = pl.pallas_call(
        _line_kernel,
        out_shape=jax.ShapeDtypeStruct((_H, 1, _LINE), jnp.float32),
    )(table)
    out = pl.pallas_call(
        _bcast_kernel,
        grid=(_H, _K // _TK),
        in_specs=[pl.BlockSpec((1, 1, _LINE), lambda h, kt: (h, 0, 0))],
        out_specs=pl.BlockSpec((_B, 1, _TK, _Q), lambda h, kt: (0, h, kt, 0)),
        out_shape=jax.ShapeDtypeStruct((_B, _H, _K, _Q), jnp.float32),
    )(gline)
    return out


# speedup vs baseline: 104.7589x; 1.0883x over previous
import math

import jax
import jax.numpy as jnp
from jax.experimental import pallas as pl
from jax.experimental.pallas import tpu as pltpu

_B = 2
_H = 16
_K = 2048
_Q = 2048
_NB = 32
_MD = 128
_LINE = 4096
_TK = 1024


def _line_kernel(table_ref, gline_ref):
    j = jax.lax.broadcasted_iota(jnp.int32, (1, _LINE), 1)
    d = 2047 - j
    nb = _NB // 2
    rb = jnp.where(d > 0, nb, 0)
    ad = jnp.abs(d)
    max_exact = nb // 2
    is_small = ad < max_exact
    rp_safe = jnp.maximum(ad, 1).astype(jnp.float32)
    if_large = max_exact + (
        jnp.log(rp_safe / max_exact) / math.log(_MD / max_exact) * (nb - max_exact)
    ).astype(jnp.int32)
    if_large = jnp.minimum(if_large, nb - 1)
    bucket = rb + jnp.where(is_small, ad, if_large)
    rows = jax.lax.broadcasted_iota(jnp.int32, (_NB, _LINE), 0)
    onehot = (rows == bucket).astype(jnp.float32)
    gline_ref[:, 0, :] = jax.lax.dot_general(
        table_ref[...], onehot, (((0,), (0,)), ((), ())),
        preferred_element_type=jnp.float32)


def _bcast_kernel(gline_ref, out_ref):
    kt = pl.program_id(1)
    big = jnp.broadcast_to(gline_ref[0], (_TK, _LINE))
    rolled = pltpu.roll(big, _LINE // 2 + 1, 1, stride=1, stride_axis=0)
    rolled = pltpu.roll(rolled, kt * _TK, 1)
    out_ref[...] = jnp.broadcast_to(rolled[None, None, :, :_Q], (_B, 1, _TK, _Q))


def kernel(batch, key_length, query_length, table):
    gline = pl.pallas_call(
        _line_kernel,
        out_shape=jax.ShapeDtypeStruct((_H, 1, _LINE), jnp.float32),
    )(table)
    out = pl.pallas_call(
        _bcast_kernel,
        grid=(_H, _K // _TK),
        in_specs=[pl.BlockSpec((1, 1, _LINE), lambda h, kt: (h, 0, 0))],
        out_specs=pl.BlockSpec((_B, 1, _TK, _Q), lambda h, kt: (0, h, kt, 0)),
        out_shape=jax.ShapeDtypeStruct((_B, _H, _K, _Q), jnp.float32),
    )(gline)
    return out


# speedup vs baseline: 109.1349x; 1.0418x over previous
import math

import jax
import jax.numpy as jnp
from jax.experimental import pallas as pl
from jax.experimental.pallas import tpu as pltpu

_B = 2
_H = 16
_K = 2048
_Q = 2048
_NB = 32
_MD = 128
_LINE = 4096
_TK = 1024


def _line_kernel(table_ref, gline_ref):
    j = jax.lax.broadcasted_iota(jnp.int32, (1, _LINE), 1)
    d = 2047 - j
    nb = _NB // 2
    rb = jnp.where(d > 0, nb, 0)
    ad = jnp.abs(d)
    max_exact = nb // 2
    is_small = ad < max_exact
    sq = (ad * ad).astype(jnp.float32)
    e = (jax.lax.bitcast_convert_type(sq, jnp.int32) >> 23) - 127
    if_large = jnp.minimum(max_exact + (e - 6), nb - 1)
    bucket = rb + jnp.where(is_small, ad, if_large)
    rows = jax.lax.broadcasted_iota(jnp.int32, (_NB, _LINE), 0)
    onehot = (rows == bucket).astype(jnp.float32)
    gline_ref[:, 0, :] = jax.lax.dot_general(
        table_ref[...], onehot, (((0,), (0,)), ((), ())),
        preferred_element_type=jnp.float32,
        precision=jax.lax.Precision.HIGHEST)


def _bcast_kernel(gline_ref, out_ref):
    kt = pl.program_id(1)
    big = jnp.broadcast_to(gline_ref[0], (_TK, _LINE))
    rolled = pltpu.roll(big, _LINE // 2 + 1, 1, stride=1, stride_axis=0)
    rolled = pltpu.roll(rolled, kt * _TK, 1)
    out_ref[...] = jnp.broadcast_to(rolled[None, None, :, :_Q], (_B, 1, _TK, _Q))


def kernel(batch, key_length, query_length, table):
    gline = pl.pallas_call(
        _line_kernel,
        out_shape=jax.ShapeDtypeStruct((_H, 1, _LINE), jnp.float32),
    )(table)
    out = pl.pallas_call(
        _bcast_kernel,
        grid=(_H, _K // _TK),
        in_specs=[pl.BlockSpec((1, 1, _LINE), lambda h, kt: (h, 0, 0))],
        out_specs=pl.BlockSpec((_B, 1, _TK, _Q), lambda h, kt: (0, h, kt, 0)),
        out_shape=jax.ShapeDtypeStruct((_B, _H, _K, _Q), jnp.float32),
        compiler_params=pltpu.CompilerParams(vmem_limit_bytes=100 * 1024 * 1024),
    )(gline)
    return out
